# baseline (speedup 1.0000x reference)
import functools

import jax
import jax.numpy as jnp
from jax import lax
from jax.experimental import pallas as pl
from jax.experimental.pallas import tpu as pltpu
from jax.experimental.pallas import tpu_sc as plsc

N = 10000
D_IN = 256
D_OUT = 256
NC, NS = 2, 16
DH = D_OUT // NC
E = 160000
CHUNK = 80
PER_TILE = E // NS
NCHUNK = PER_TILE // CHUNK
SUPER = 25
NSUP = NCHUNK // SUPER
SUP_E = SUPER * CHUNK
NP = 10240
ROW_T = NP // NS
CP = 80
LANES = 16
GR = 8
NBUF = 3


def _mm_body(x_ref, w_ref, o_ref):
    o_ref[...] = jnp.dot(x_ref[...], w_ref[...],
                         preferred_element_type=jnp.float32)


def _matmul_split(x, w):
    bm = 1000
    nm = N // bm
    return pl.pallas_call(
        _mm_body,
        grid=(NC, nm),
        in_specs=[
            pl.BlockSpec((bm, D_IN), lambda c, m: (m, 0)),
            pl.BlockSpec((D_IN, DH), lambda c, m: (0, c)),
        ],
        out_specs=pl.BlockSpec((bm, DH), lambda c, m: (c * nm + m, 0)),
        out_shape=jax.ShapeDtypeStruct((NC * N, DH), jnp.float32),
    )(x, w)


def _spmm_body(sup_hbm, ei_hbm, ew_hbm, out_hbm,
               sidx0, sidx1, didx0, didx1, eww0, eww1, didc, rows, acc,
               sg0, sg1, sg2, ss0, ss1, ss2,
               ki0, ki1, kd0, kd1, kw0, kw1):
    sg = (sg0, sg1, sg2)
    ss = (ss0, ss1, ss2)
    ki = (ki0, ki1)
    kd = (kd0, kd1)
    kw = (kw0, kw1)
    sidx = (sidx0, sidx1)
    didx = (didx0, didx1)
    eww = (eww0, eww1)
    c = lax.axis_index("c")
    s = lax.axis_index("s")
    coff = c * N

    @pl.loop(0, CP)
    def _zero(r):
        for k in range(DH // LANES):
            rows[0, r, pl.ds(k * LANES, LANES)] = jnp.zeros((LANES,),
                                                            jnp.float32)

    for j in range(ROW_T // CP):
        pltpu.sync_copy(rows.at[0],
                        acc.at[pl.ds(s * ROW_T + j * CP, CP)])
    plsc.subcore_barrier()

    def issue_super(u, p):
        base = s * PER_TILE + u * SUP_E
        pltpu.async_copy(ei_hbm.at[pl.ds(base, SUP_E)], sidx[p], ki[p])
        pltpu.async_copy(ei_hbm.at[pl.ds(E + base, SUP_E)], didx[p],
                         kd[p])
        pltpu.async_copy(ew_hbm.at[pl.ds(base, SUP_E)],
                         eww[p].at[pl.ds(0, SUP_E)], kw[p])

    def wait_super(p):
        pltpu.make_async_copy(ei_hbm.at[pl.ds(0, SUP_E)], sidx[p],
                              ki[p]).wait()
        pltpu.make_async_copy(ei_hbm.at[pl.ds(0, SUP_E)], didx[p],
                              kd[p]).wait()
        pltpu.make_async_copy(ew_hbm.at[pl.ds(0, SUP_E)],
                              eww[p].at[pl.ds(0, SUP_E)], kw[p]).wait()

    def prep(bn, p, jj):
        base = jj * CHUNK
        for g in range(CHUNK // LANES):
            sl = pl.ds(base + g * LANES, LANES)
            sidx[p][sl] = sidx[p][sl] + coff
            didc[bn, pl.ds(g * LANES, LANES)] = didx[p][sl]
        pltpu.async_copy(sup_hbm.at[sidx[p].at[pl.ds(base, CHUNK)]],
                         rows.at[bn], sg[bn])

    def wait_gather(b):
        pltpu.make_async_copy(sup_hbm.at[pl.ds(0, CHUNK)], rows.at[b],
                              sg[b]).wait()

    def scale(b, p, jj):
        @pl.loop(0, CHUNK // GR)
        def _sc(g):
            wv = eww[p][pl.ds(jj * CHUNK + g * GR, LANES)]
            for t in range(GR):
                w = wv[t]
                e = g * GR + t
                for k in range(DH // LANES):
                    sl = pl.ds(k * LANES, LANES)
                    rows[b, e, sl] = rows[b, e, sl] * w

    def issue_scatter(b):
        pltpu.async_copy(rows.at[b], acc.at[didc.at[b]], ss[b], add=True)

    def wait_scatter(b):
        pltpu.make_async_copy(sup_hbm.at[pl.ds(0, CHUNK)], rows.at[b],
                              ss[b]).wait()

    def proc(b, p, jj):
        wait_gather(b)
        scale(b, p, jj)
        issue_scatter(b)

    pltpu.async_copy(sup_hbm.at[pl.ds(0, CP)], rows.at[1], ss1)
    pltpu.async_copy(sup_hbm.at[pl.ds(0, CP)], rows.at[2], ss2)
    issue_super(0, 0)
    wait_super(0)
    prep(0, 0, 0)
    issue_super(1, 1)

    for u in range(NSUP):
        p = u % 2

        @pl.loop(0, SUPER - 1, step=NBUF)
        def _inner(j):
            for k in range(NBUF):
                b = (u + k) % NBUF
                bn = (u + k + 1) % NBUF
                jj = j + k
                wait_scatter(bn)
                prep(bn, p, jj + 1)
                if u >= 1 and u + 1 < NSUP and k == NBUF - 1:
                    @pl.when(j == 0)
                    def _pf():
                        issue_super(u + 1, (u + 1) % 2)
                proc(b, p, jj)

        b = u % NBUF
        bn = (u + 1) % NBUF
        if u + 1 < NSUP:
            wait_scatter(bn)
            wait_super((u + 1) % 2)
            prep(bn, (u + 1) % 2, 0)
        proc(b, p, SUPER - 1)

    for b in range(NBUF):
        wait_scatter(b)

    plsc.subcore_barrier()

    nout = jnp.where(s == NS - 1, (N - (NS - 1) * ROW_T) // CP, ROW_T // CP)
    col0 = pl.multiple_of(c * DH, 128)

    @pl.loop(0, nout)
    def _out(j):
        r0 = pl.multiple_of(s * ROW_T + j * CP, 8)
        pltpu.sync_copy(acc.at[pl.ds(r0, CP)],
                        out_hbm.at[pl.ds(r0, CP), pl.ds(col0, DH)])


def _spmm_sc(sup, ei, ew):
    mesh = plsc.VectorSubcoreMesh(core_axis_name="c", subcore_axis_name="s",
                                  num_cores=NC, num_subcores=NS)
    run = pl.kernel(
        _spmm_body,
        out_type=jax.ShapeDtypeStruct((N, D_OUT), jnp.float32),
        mesh=mesh,
        scratch_types=[
            pltpu.VMEM((SUP_E,), jnp.int32),
            pltpu.VMEM((SUP_E,), jnp.int32),
            pltpu.VMEM((SUP_E,), jnp.int32),
            pltpu.VMEM((SUP_E,), jnp.int32),
            pltpu.VMEM((SUP_E + LANES,), jnp.float32),
            pltpu.VMEM((SUP_E + LANES,), jnp.float32),
            pltpu.VMEM((NBUF, CHUNK), jnp.int32),
            pltpu.VMEM((NBUF, CHUNK, DH), jnp.float32),
            pltpu.VMEM_SHARED((NP, DH), jnp.float32),
        ] + [pltpu.SemaphoreType.DMA] * 12,
    )
    return run(sup, ei, ew)


def kernel(input, edge_index, edge_weight, W):
    ei = edge_index.astype(jnp.int32).reshape(2 * E)
    sup = _matmul_split(input, W)
    return _spmm_sc(sup, ei, edge_weight)
